# baseline (device time: 45959 ns/iter reference)
import jax
import jax.numpy as jnp
from jax import lax
from jax.experimental import pallas as pl
from jax.experimental.pallas import tpu as pltpu

N_DEV = 4


def _gelu(y):
    c = 0.7978845608028654
    return 0.5 * y * (1.0 + jnp.tanh(c * (y + 0.044715 * y * y * y)))


def kernel(x, w_mat):
    m_per, k = x.shape
    _, n_per = w_mat.shape

    def body(x_ref, w_ref, out_ref, comm_ref, send_sems, recv_sems):
        my_pos = lax.axis_index("i")
        left = lax.rem(my_pos + (N_DEV - 1), N_DEV)
        right = lax.rem(my_pos + 1, N_DEV)

        barrier_sem = pltpu.get_barrier_semaphore()
        for nbr in [left, right]:
            pl.semaphore_signal(
                barrier_sem, inc=1,
                device_id=(nbr,), device_id_type=pl.DeviceIdType.MESH,
            )
        pl.semaphore_wait(barrier_sem, 2)

        comm_ref[0] = x_ref[...]

        for h in range(N_DEV - 1):
            send_slot = h % 2
            recv_slot = (h + 1) % 2
            rdma = pltpu.make_async_remote_copy(
                src_ref=comm_ref.at[send_slot],
                dst_ref=comm_ref.at[recv_slot],
                send_sem=send_sems.at[send_slot],
                recv_sem=recv_sems.at[recv_slot],
                device_id=(right,),
                device_id_type=pl.DeviceIdType.MESH,
            )
            rdma.start()
            origin = lax.rem(my_pos + (N_DEV - h), N_DEV)
            out_ref[pl.ds(origin * m_per, m_per), :] = _gelu(
                jnp.dot(
                    comm_ref[send_slot], w_ref[...],
                    preferred_element_type=jnp.float32,
                )
            )
            rdma.wait()

        origin = lax.rem(my_pos + 1, N_DEV)
        out_ref[pl.ds(origin * m_per, m_per), :] = _gelu(
            jnp.dot(
                comm_ref[(N_DEV - 1) % 2], w_ref[...],
                preferred_element_type=jnp.float32,
            )
        )

    return pl.pallas_call(
        body,
        out_shape=jax.ShapeDtypeStruct((N_DEV * m_per, n_per), jnp.float32),
        in_specs=[
            pl.BlockSpec(memory_space=pltpu.VMEM),
            pl.BlockSpec(memory_space=pltpu.VMEM),
        ],
        out_specs=pl.BlockSpec(memory_space=pltpu.VMEM),
        scratch_shapes=[
            pltpu.VMEM((2, m_per, k), jnp.float32),
            pltpu.SemaphoreType.DMA((2,)),
            pltpu.SemaphoreType.DMA((2,)),
        ],
        compiler_params=pltpu.CompilerParams(collective_id=0),
    )(x, w_mat)


# device time: 27462 ns/iter; 1.6735x vs baseline; 1.6735x over previous
import jax
import jax.numpy as jnp
from jax import lax
from jax.experimental import pallas as pl
from jax.experimental.pallas import tpu as pltpu

N_DEV = 4


def _gelu(y):
    c = 0.7978845608028654
    return 0.5 * y * (1.0 + jnp.tanh(c * (y + 0.044715 * y * y * y)))


def kernel(x, w_mat):
    m_per, k = x.shape
    _, n_per = w_mat.shape
    half = m_per // 2

    def body(x_ref, w_ref, out_ref, buf_l, buf_r, buf_o, send_sems, recv_sems):
        my_pos = lax.axis_index("i")
        left = lax.rem(my_pos + (N_DEV - 1), N_DEV)
        right = lax.rem(my_pos + 1, N_DEV)

        barrier_sem = pltpu.get_barrier_semaphore()
        for nbr in [left, right]:
            pl.semaphore_signal(
                barrier_sem, inc=1,
                device_id=(nbr,), device_id_type=pl.DeviceIdType.MESH,
            )
        pl.semaphore_wait(barrier_sem, 2)

        rdma_r = pltpu.make_async_remote_copy(
            src_ref=x_ref, dst_ref=buf_l,
            send_sem=send_sems.at[0], recv_sem=recv_sems.at[0],
            device_id=(right,), device_id_type=pl.DeviceIdType.MESH,
        )
        rdma_l = pltpu.make_async_remote_copy(
            src_ref=x_ref, dst_ref=buf_r,
            send_sem=send_sems.at[1], recv_sem=recv_sems.at[1],
            device_id=(left,), device_id_type=pl.DeviceIdType.MESH,
        )
        rdma_r.start()
        rdma_l.start()

        out_ref[pl.ds(my_pos * m_per, m_per), :] = _gelu(
            jnp.dot(x_ref[...], w_ref[...], preferred_element_type=jnp.float32)
        )

        rdma_r.wait_recv()
        rdma_r2 = pltpu.make_async_remote_copy(
            src_ref=buf_l.at[pl.ds(0, half), :],
            dst_ref=buf_o.at[pl.ds(0, half), :],
            send_sem=send_sems.at[2], recv_sem=recv_sems.at[2],
            device_id=(right,), device_id_type=pl.DeviceIdType.MESH,
        )
        rdma_r2.start()
        origin = lax.rem(my_pos + (N_DEV - 1), N_DEV)
        out_ref[pl.ds(origin * m_per, m_per), :] = _gelu(
            jnp.dot(buf_l[...], w_ref[...], preferred_element_type=jnp.float32)
        )

        rdma_l.wait_recv()
        rdma_l2 = pltpu.make_async_remote_copy(
            src_ref=buf_r.at[pl.ds(half, half), :],
            dst_ref=buf_o.at[pl.ds(half, half), :],
            send_sem=send_sems.at[3], recv_sem=recv_sems.at[3],
            device_id=(left,), device_id_type=pl.DeviceIdType.MESH,
        )
        rdma_l2.start()
        origin = lax.rem(my_pos + 1, N_DEV)
        out_ref[pl.ds(origin * m_per, m_per), :] = _gelu(
            jnp.dot(buf_r[...], w_ref[...], preferred_element_type=jnp.float32)
        )

        rdma_r2.wait_recv()
        rdma_l2.wait_recv()
        origin = lax.rem(my_pos + 2, N_DEV)
        out_ref[pl.ds(origin * m_per, m_per), :] = _gelu(
            jnp.dot(buf_o[...], w_ref[...], preferred_element_type=jnp.float32)
        )

        rdma_r.wait_send()
        rdma_l.wait_send()
        rdma_r2.wait_send()
        rdma_l2.wait_send()

    return pl.pallas_call(
        body,
        out_shape=jax.ShapeDtypeStruct((N_DEV * m_per, n_per), jnp.float32),
        in_specs=[
            pl.BlockSpec(memory_space=pltpu.VMEM),
            pl.BlockSpec(memory_space=pltpu.VMEM),
        ],
        out_specs=pl.BlockSpec(memory_space=pltpu.VMEM),
        scratch_shapes=[
            pltpu.VMEM((m_per, k), jnp.float32),
            pltpu.VMEM((m_per, k), jnp.float32),
            pltpu.VMEM((m_per, k), jnp.float32),
            pltpu.SemaphoreType.DMA((4,)),
            pltpu.SemaphoreType.DMA((4,)),
        ],
        compiler_params=pltpu.CompilerParams(collective_id=0),
    )(x, w_mat)


# device time: 25774 ns/iter; 1.7832x vs baseline; 1.0655x over previous
import jax
import jax.numpy as jnp
from jax import lax
from jax.experimental import pallas as pl
from jax.experimental.pallas import tpu as pltpu

N_DEV = 4


def _gelu(y):
    c = 0.7978845608028654
    return 0.5 * y * (1.0 + jnp.tanh(c * (y + 0.044715 * y * y * y)))


def kernel(x, w_mat):
    m_per, k = x.shape
    _, n_per = w_mat.shape
    h = m_per // 2

    def body(x_ref, w_ref, out_ref, buf_l, buf_r, buf_o, send_sems, recv_sems):
        my_pos = lax.axis_index("i")
        left = lax.rem(my_pos + (N_DEV - 1), N_DEV)
        right = lax.rem(my_pos + 1, N_DEV)

        def copy(src, dst, sem_idx, target):
            return pltpu.make_async_remote_copy(
                src_ref=src, dst_ref=dst,
                send_sem=send_sems.at[sem_idx], recv_sem=recv_sems.at[sem_idx],
                device_id=(target,), device_id_type=pl.DeviceIdType.MESH,
            )

        barrier_sem = pltpu.get_barrier_semaphore()
        for nbr in [left, right]:
            pl.semaphore_signal(
                barrier_sem, inc=1,
                device_id=(nbr,), device_id_type=pl.DeviceIdType.MESH,
            )
        pl.semaphore_wait(barrier_sem, 2)

        r1a = copy(x_ref.at[pl.ds(0, h), :], buf_l.at[pl.ds(0, h), :], 0, right)
        r1b = copy(x_ref.at[pl.ds(h, h), :], buf_l.at[pl.ds(h, h), :], 1, right)
        l1a = copy(x_ref.at[pl.ds(h, h), :], buf_r.at[pl.ds(h, h), :], 2, left)
        l1b = copy(x_ref.at[pl.ds(0, h), :], buf_r.at[pl.ds(0, h), :], 3, left)
        r1a.start()
        l1a.start()
        r1b.start()
        l1b.start()

        out_ref[pl.ds(my_pos * m_per, m_per), :] = _gelu(
            jnp.dot(x_ref[...], w_ref[...], preferred_element_type=jnp.float32)
        )

        r1a.wait_recv()
        r2 = copy(buf_l.at[pl.ds(0, h), :], buf_o.at[pl.ds(0, h), :], 4, right)
        r2.start()
        l1a.wait_recv()
        l2 = copy(buf_r.at[pl.ds(h, h), :], buf_o.at[pl.ds(h, h), :], 5, left)
        l2.start()

        r1b.wait_recv()
        origin = lax.rem(my_pos + (N_DEV - 1), N_DEV)
        out_ref[pl.ds(origin * m_per, m_per), :] = _gelu(
            jnp.dot(buf_l[...], w_ref[...], preferred_element_type=jnp.float32)
        )
        l1b.wait_recv()
        origin = lax.rem(my_pos + 1, N_DEV)
        out_ref[pl.ds(origin * m_per, m_per), :] = _gelu(
            jnp.dot(buf_r[...], w_ref[...], preferred_element_type=jnp.float32)
        )

        r2.wait_recv()
        l2.wait_recv()
        origin = lax.rem(my_pos + 2, N_DEV)
        out_ref[pl.ds(origin * m_per, m_per), :] = _gelu(
            jnp.dot(buf_o[...], w_ref[...], preferred_element_type=jnp.float32)
        )

        for rdma in (r1a, r1b, l1a, l1b, r2, l2):
            rdma.wait_send()

    return pl.pallas_call(
        body,
        out_shape=jax.ShapeDtypeStruct((N_DEV * m_per, n_per), jnp.float32),
        in_specs=[
            pl.BlockSpec(memory_space=pltpu.VMEM),
            pl.BlockSpec(memory_space=pltpu.VMEM),
        ],
        out_specs=pl.BlockSpec(memory_space=pltpu.VMEM),
        scratch_shapes=[
            pltpu.VMEM((m_per, k), jnp.float32),
            pltpu.VMEM((m_per, k), jnp.float32),
            pltpu.VMEM((m_per, k), jnp.float32),
            pltpu.SemaphoreType.DMA((6,)),
            pltpu.SemaphoreType.DMA((6,)),
        ],
        compiler_params=pltpu.CompilerParams(collective_id=0),
    )(x, w_mat)
